# Initial kernel scaffold; baseline (speedup 1.0000x reference)
import jax
import jax.numpy as jnp
from jax.experimental import pallas as pl


def kernel(features, edge_index, edge_vals, W0, W1, W2, att_w, att_b, fc_w, fc_b):
    raise NotImplementedError("write your pallas kernel here")


# speedup vs baseline: 1.0551x; 1.0551x over previous
import functools

import jax
import jax.numpy as jnp
from jax.experimental import pallas as pl

N = 10000
E = 320000
NFEAT = 128
NHID = 128
NCLASS = 64
HOPS = 3

ROW_BLK = 1000


def _hop_body(agg_ref, w_ref, out_ref):
    out_ref[...] = jnp.maximum(
        jnp.dot(agg_ref[...], w_ref[...], preferred_element_type=jnp.float32), 0.0
    )


def _tc_hop(agg, W):
    return pl.pallas_call(
        _hop_body,
        grid=(N // ROW_BLK,),
        in_specs=[
            pl.BlockSpec((ROW_BLK, NHID), lambda i: (i, 0)),
            pl.BlockSpec((NHID, NHID), lambda i: (0, 0)),
        ],
        out_specs=pl.BlockSpec((ROW_BLK, NHID), lambda i: (i, 0)),
        out_shape=jax.ShapeDtypeStruct((N, NHID), jnp.float32),
    )(agg, W)


def _final_body(h1_ref, h2_ref, h3_ref, x_ref, ax_ref, af_ref, ab_ref,
                fcw_ref, fcb_ref, out_ref):
    x = x_ref[...]
    c = jnp.sum(x * af_ref[...], axis=1, keepdims=True) + ab_ref[0, 0]
    ax = ax_ref[...]
    h1 = h1_ref[...]
    h2 = h2_ref[...]
    h3 = h3_ref[...]
    s1 = jax.nn.sigmoid(jnp.sum(h1 * ax, axis=1, keepdims=True) + c)
    s2 = jax.nn.sigmoid(jnp.sum(h2 * ax, axis=1, keepdims=True) + c)
    s3 = jax.nn.sigmoid(jnp.sum(h3 * ax, axis=1, keepdims=True) + c)
    m = jnp.maximum(jnp.maximum(s1, s2), s3)
    e1 = jnp.exp(s1 - m)
    e2 = jnp.exp(s2 - m)
    e3 = jnp.exp(s3 - m)
    denom = e1 + e2 + e3
    comb = h1 * ((e1 + e2) / denom) + h2 * (e3 / denom)
    o = jnp.dot(comb, fcw_ref[...], preferred_element_type=jnp.float32) + fcb_ref[...]
    mx = jnp.max(o, axis=1, keepdims=True)
    lse = jnp.log(jnp.sum(jnp.exp(o - mx), axis=1, keepdims=True)) + mx
    out_ref[...] = o - lse


def _tc_final(h1, h2, h3, x, ax, af, ab, fc_w, fc_b):
    row = lambda i: (i, 0)
    full = lambda i: (0, 0)
    return pl.pallas_call(
        _final_body,
        grid=(N // ROW_BLK,),
        in_specs=[
            pl.BlockSpec((ROW_BLK, NHID), row),
            pl.BlockSpec((ROW_BLK, NHID), row),
            pl.BlockSpec((ROW_BLK, NHID), row),
            pl.BlockSpec((ROW_BLK, NFEAT), row),
            pl.BlockSpec((1, NHID), full),
            pl.BlockSpec((1, NFEAT), full),
            pl.BlockSpec((1, 1), full),
            pl.BlockSpec((NHID, NCLASS), full),
            pl.BlockSpec((1, NCLASS), full),
        ],
        out_specs=pl.BlockSpec((ROW_BLK, NCLASS), row),
        out_shape=jax.ShapeDtypeStruct((N, NCLASS), jnp.float32),
    )(h1, h2, h3, x, ax, af, ab, fc_w, fc_b)


def kernel(features, edge_index, edge_vals, W0, W1, W2, att_w, att_b, fc_w, fc_b):
    dst = edge_index[0]
    src = edge_index[1]
    Ws = [W0, W1, W2]
    hs = []
    h = features
    for i in range(HOPS):
        msgs = edge_vals[:, None] * jnp.take(h, src, axis=0)
        agg = jax.ops.segment_sum(msgs, dst, num_segments=N)
        h = _tc_hop(agg, Ws[i])
        hs.append(h)
    ax = att_w[:NHID, 0].reshape(1, NHID)
    af = att_w[NHID:, 0].reshape(1, NFEAT)
    ab = att_b.reshape(1, 1)
    return _tc_final(hs[0], hs[1], hs[2], features, ax, af, ab,
                     fc_w, fc_b.reshape(1, NCLASS))


# speedup vs baseline: 4.0820x; 3.8690x over previous
import functools

import jax
import jax.numpy as jnp
from jax import lax
from jax.experimental import pallas as pl
from jax.experimental.pallas import tpu as pltpu
from jax.experimental.pallas import tpu_sc as plsc

N = 10000
E = 320000
NFEAT = 128
NHID = 128
NCLASS = 64
HOPS = 3

ROW_BLK = 1000

NC = 2
NS = 16
NW = NC * NS

K = 80
EDGES_PER_W = E // NW
CHUNKS_PER_W = EDGES_PER_W // K
ZBLK = 128

STRIPE = 640

_GATHER_DNUMS = lax.GatherDimensionNumbers(
    offset_dims=(), collapsed_slice_dims=(0,), start_index_map=(0,))


def _bcast_lane(vec, ii):
    idx = jnp.full((16, 1), ii, jnp.int32)
    return lax.gather(vec, idx, _GATHER_DNUMS, (1,),
                      mode=lax.GatherScatterMode.PROMISE_IN_BOUNDS)


def _spmm_sc(h, src, dst, ev):
    mesh = plsc.VectorSubcoreMesh(
        core_axis_name="c", subcore_axis_name="s", num_cores=NC, num_subcores=NS
    )

    @functools.partial(
        pl.kernel,
        out_type=(jax.ShapeDtypeStruct((N, NHID), jnp.float32),
                  jax.ShapeDtypeStruct((N, NHID), jnp.float32)),
        mesh=mesh,
        scratch_types=[
            pltpu.VMEM((K,), jnp.int32),
            pltpu.VMEM((K,), jnp.int32),
            pltpu.VMEM((K,), jnp.float32),
            pltpu.VMEM((K, NHID), jnp.float32),
            pltpu.VMEM((ZBLK, NHID), jnp.float32),
            pltpu.VMEM_SHARED((N, NHID), jnp.float32),
            pltpu.SemaphoreType.DMA,
        ],
    )
    def spmm_kernel(h_hbm, src_hbm, dst_hbm, ev_hbm, out0_hbm, out1_hbm,
                    src_v, dst_v, ev_v, rows_v, zero_v, agg_sh, sem):
        cid = lax.axis_index("c")
        sid = lax.axis_index("s")
        wid = sid * NC + cid

        zeros16 = jnp.zeros((16,), jnp.float32)

        def zrow(i, carry):
            for j in range(NHID // 16):
                zero_v[i, pl.ds(j * 16, 16)] = zeros16
            return carry

        lax.fori_loop(0, ZBLK, zrow, 0)
        base = sid * STRIPE

        @pl.when(sid < NS - 1)
        def _():
            for j in range(STRIPE // ZBLK):
                pltpu.sync_copy(zero_v, agg_sh.at[pl.ds(base + j * ZBLK, ZBLK)])

        @pl.when(sid == NS - 1)
        def _():
            for j in range(3):
                pltpu.sync_copy(zero_v, agg_sh.at[pl.ds(base + j * ZBLK, ZBLK)])
            pltpu.sync_copy(zero_v.at[pl.ds(0, 16)],
                            agg_sh.at[pl.ds(base + 3 * ZBLK, 16)])

        plsc.subcore_barrier()

        ebase = wid * EDGES_PER_W

        def chunk(t, carry):
            off = ebase + t * K
            pltpu.sync_copy(src_hbm.at[pl.ds(off, K)], src_v)
            pltpu.sync_copy(dst_hbm.at[pl.ds(off, K)], dst_v)
            pltpu.sync_copy(ev_hbm.at[pl.ds(off, K)], ev_v)
            pltpu.async_copy(h_hbm.at[src_v], rows_v, sem).wait()

            def grp(g, rcarry):
                evs = ev_v[pl.ds(g * 16, 16)]
                for ii in range(16):
                    e = _bcast_lane(evs, ii)
                    r = g * 16 + ii
                    for j in range(NHID // 16):
                        s = pl.ds(j * 16, 16)
                        rows_v[r, s] = rows_v[r, s] * e
                return rcarry

            lax.fori_loop(0, K // 16, grp, 0)
            pltpu.sync_copy(rows_v, agg_sh.at[dst_v], add=True)
            return carry

        lax.fori_loop(0, CHUNKS_PER_W, chunk, 0)
        plsc.subcore_barrier()

        TAIL = N - (NS - 1) * STRIPE

        @pl.when(jnp.logical_and(cid == 0, sid < NS - 1))
        def _():
            pltpu.sync_copy(agg_sh.at[pl.ds(base, STRIPE)],
                            out0_hbm.at[pl.ds(base, STRIPE)])

        @pl.when(jnp.logical_and(cid == 0, sid == NS - 1))
        def _():
            pltpu.sync_copy(agg_sh.at[pl.ds(base, TAIL)],
                            out0_hbm.at[pl.ds(base, TAIL)])

        @pl.when(jnp.logical_and(cid == 1, sid < NS - 1))
        def _():
            pltpu.sync_copy(agg_sh.at[pl.ds(base, STRIPE)],
                            out1_hbm.at[pl.ds(base, STRIPE)])

        @pl.when(jnp.logical_and(cid == 1, sid == NS - 1))
        def _():
            pltpu.sync_copy(agg_sh.at[pl.ds(base, TAIL)],
                            out1_hbm.at[pl.ds(base, TAIL)])

    return spmm_kernel(h, src, dst, ev)


def _hop_body(a0_ref, a1_ref, w_ref, out_ref):
    a = a0_ref[...] + a1_ref[...]
    out_ref[...] = jnp.maximum(
        jnp.dot(a, w_ref[...], preferred_element_type=jnp.float32), 0.0
    )


def _tc_hop(a0, a1, W):
    return pl.pallas_call(
        _hop_body,
        grid=(N // ROW_BLK,),
        in_specs=[
            pl.BlockSpec((ROW_BLK, NHID), lambda i: (i, 0)),
            pl.BlockSpec((ROW_BLK, NHID), lambda i: (i, 0)),
            pl.BlockSpec((NHID, NHID), lambda i: (0, 0)),
        ],
        out_specs=pl.BlockSpec((ROW_BLK, NHID), lambda i: (i, 0)),
        out_shape=jax.ShapeDtypeStruct((N, NHID), jnp.float32),
    )(a0, a1, W)


def _final_body(h1_ref, h2_ref, h3_ref, x_ref, ax_ref, af_ref, ab_ref,
                fcw_ref, fcb_ref, out_ref):
    x = x_ref[...]
    c = jnp.sum(x * af_ref[...], axis=1, keepdims=True) + ab_ref[0, 0]
    ax = ax_ref[...]
    h1 = h1_ref[...]
    h2 = h2_ref[...]
    h3 = h3_ref[...]
    s1 = jax.nn.sigmoid(jnp.sum(h1 * ax, axis=1, keepdims=True) + c)
    s2 = jax.nn.sigmoid(jnp.sum(h2 * ax, axis=1, keepdims=True) + c)
    s3 = jax.nn.sigmoid(jnp.sum(h3 * ax, axis=1, keepdims=True) + c)
    m = jnp.maximum(jnp.maximum(s1, s2), s3)
    e1 = jnp.exp(s1 - m)
    e2 = jnp.exp(s2 - m)
    e3 = jnp.exp(s3 - m)
    denom = e1 + e2 + e3
    comb = h1 * ((e1 + e2) / denom) + h2 * (e3 / denom)
    o = jnp.dot(comb, fcw_ref[...], preferred_element_type=jnp.float32) + fcb_ref[...]
    mx = jnp.max(o, axis=1, keepdims=True)
    lse = jnp.log(jnp.sum(jnp.exp(o - mx), axis=1, keepdims=True)) + mx
    out_ref[...] = o - lse


def _tc_final(h1, h2, h3, x, ax, af, ab, fc_w, fc_b):
    row = lambda i: (i, 0)
    full = lambda i: (0, 0)
    return pl.pallas_call(
        _final_body,
        grid=(N // ROW_BLK,),
        in_specs=[
            pl.BlockSpec((ROW_BLK, NHID), row),
            pl.BlockSpec((ROW_BLK, NHID), row),
            pl.BlockSpec((ROW_BLK, NHID), row),
            pl.BlockSpec((ROW_BLK, NFEAT), row),
            pl.BlockSpec((1, NHID), full),
            pl.BlockSpec((1, NFEAT), full),
            pl.BlockSpec((1, 1), full),
            pl.BlockSpec((NHID, NCLASS), full),
            pl.BlockSpec((1, NCLASS), full),
        ],
        out_specs=pl.BlockSpec((ROW_BLK, NCLASS), row),
        out_shape=jax.ShapeDtypeStruct((N, NCLASS), jnp.float32),
    )(h1, h2, h3, x, ax, af, ab, fc_w, fc_b)


def kernel(features, edge_index, edge_vals, W0, W1, W2, att_w, att_b, fc_w, fc_b):
    dst = edge_index[0]
    src = edge_index[1]
    Ws = [W0, W1, W2]
    hs = []
    h = features
    for i in range(HOPS):
        a0, a1 = _spmm_sc(h, src, dst, edge_vals)
        h = _tc_hop(a0, a1, Ws[i])
        hs.append(h)
    ax = att_w[:NHID, 0].reshape(1, NHID)
    af = att_w[NHID:, 0].reshape(1, NFEAT)
    ab = att_b.reshape(1, 1)
    return _tc_final(hs[0], hs[1], hs[2], features, ax, af, ab,
                     fc_w, fc_b.reshape(1, NCLASS))


# speedup vs baseline: 9.5516x; 2.3399x over previous
import functools

import jax
import jax.numpy as jnp
from jax import lax
from jax.experimental import pallas as pl
from jax.experimental.pallas import tpu as pltpu
from jax.experimental.pallas import tpu_sc as plsc

N = 10000
E = 320000
NFEAT = 128
NHID = 128
NCLASS = 64
HOPS = 3

ROW_BLK = 1000

NC = 2
NS = 16
NW = NC * NS

K = 80
EDGES_PER_W = E // NW
CHUNKS_PER_W = EDGES_PER_W // K
ZBLK = 64
NBUF = 3

STRIPE = 640

_GATHER_DNUMS = lax.GatherDimensionNumbers(
    offset_dims=(), collapsed_slice_dims=(0,), start_index_map=(0,))


def _bcast_lane(vec, ii):
    idx = jnp.full((16, 1), ii, jnp.int32)
    return lax.gather(vec, idx, _GATHER_DNUMS, (1,),
                      mode=lax.GatherScatterMode.PROMISE_IN_BOUNDS)


def _spmm_sc(h, src2, dst2, ev2):
    mesh = plsc.VectorSubcoreMesh(
        core_axis_name="c", subcore_axis_name="s", num_cores=NC, num_subcores=NS
    )

    @functools.partial(
        pl.kernel,
        out_type=(jax.ShapeDtypeStruct((N, NHID), jnp.float32),
                  jax.ShapeDtypeStruct((N, NHID), jnp.float32)),
        mesh=mesh,
        scratch_types=[
            pltpu.VMEM((NBUF, 1, K), jnp.int32),
            pltpu.VMEM((NBUF, 1, K), jnp.int32),
            pltpu.VMEM((NBUF, 1, K), jnp.float32),
            pltpu.VMEM((NBUF, K, NHID), jnp.float32),
            pltpu.VMEM((ZBLK, NHID), jnp.float32),
            pltpu.VMEM_SHARED((N, NHID), jnp.float32),
            [pltpu.SemaphoreType.DMA] * NBUF,
            [pltpu.SemaphoreType.DMA] * NBUF,
            [pltpu.SemaphoreType.DMA] * NBUF,
        ],
    )
    def spmm_kernel(h_hbm, src_hbm, dst_hbm, ev_hbm, out0_hbm, out1_hbm,
                    src_v, dst_v, ev_v, rows_v, zero_v, agg_sh,
                    isem, gsem, ssem):
        cid = lax.axis_index("c")
        sid = lax.axis_index("s")
        wid = sid * NC + cid
        cbase = wid * CHUNKS_PER_W

        zeros16 = jnp.zeros((16,), jnp.float32)

        def zrow(i, carry):
            for j in range(NHID // 16):
                zero_v[i, pl.ds(j * 16, 16)] = zeros16
            return carry

        lax.fori_loop(0, ZBLK, zrow, 0)
        base = sid * STRIPE

        @pl.when(sid < NS - 1)
        def _():
            for j in range(STRIPE // ZBLK):
                pltpu.sync_copy(zero_v, agg_sh.at[pl.ds(base + j * ZBLK, ZBLK)])

        @pl.when(sid == NS - 1)
        def _():
            for j in range(6):
                pltpu.sync_copy(zero_v, agg_sh.at[pl.ds(base + j * ZBLK, ZBLK)])
            pltpu.sync_copy(zero_v.at[pl.ds(0, 16)],
                            agg_sh.at[pl.ds(base + 6 * ZBLK, 16)])

        plsc.subcore_barrier()

        def load_idx(c, q):
            row = cbase + c
            pltpu.async_copy(src_hbm.at[row], src_v.at[q], isem[q])
            pltpu.async_copy(dst_hbm.at[row], dst_v.at[q], isem[q])
            pltpu.async_copy(ev_hbm.at[row], ev_v.at[q], isem[q])

        def wait_idx(c, q):
            row = cbase + c
            pltpu.make_async_copy(src_hbm.at[row], src_v.at[q], isem[q]).wait()
            pltpu.make_async_copy(dst_hbm.at[row], dst_v.at[q], isem[q]).wait()
            pltpu.make_async_copy(ev_hbm.at[row], ev_v.at[q], isem[q]).wait()

        def start_gather(c, b):
            pltpu.async_copy(h_hbm.at[src_v.at[b, 0]], rows_v.at[b], gsem[b])

        def wait_gather(c, b):
            pltpu.make_async_copy(h_hbm.at[src_v.at[b, 0]], rows_v.at[b],
                                  gsem[b]).wait()

        def scale(c, b):
            def grp(g, rcarry):
                evs = ev_v[b, 0, pl.ds(g * 16, 16)]
                for ii in range(16):
                    e = _bcast_lane(evs, ii)
                    r = g * 16 + ii
                    for j in range(NHID // 16):
                        sl = pl.ds(j * 16, 16)
                        rows_v[b, r, sl] = rows_v[b, r, sl] * e
                return rcarry

            lax.fori_loop(0, K // 16, grp, 0)

        def start_scatter(c, b):
            pltpu.async_copy(rows_v.at[b], agg_sh.at[dst_v.at[b, 0]],
                             ssem[b], add=True)

        def wait_scatter(c, b):
            pltpu.make_async_copy(rows_v.at[b], agg_sh.at[dst_v.at[b, 0]],
                                  ssem[b]).wait()

        load_idx(0, 0)
        load_idx(1, 1)
        load_idx(2, 2)
        wait_idx(0, 0)
        start_gather(0, 0)
        wait_gather(0, 0)
        wait_idx(1, 1)
        start_gather(1, 1)
        scale(0, 0)
        start_scatter(0, 0)
        wait_gather(1, 1)
        wait_idx(2, 2)
        start_gather(2, 2)
        scale(1, 1)
        wait_scatter(0, 0)
        start_scatter(1, 1)
        load_idx(3, 0)
        wait_gather(2, 2)
        wait_idx(3, 0)
        start_gather(3, 0)
        scale(2, 2)
        wait_scatter(1, 1)
        start_scatter(2, 2)
        load_idx(4, 1)

        def step(t, carry):
            c0 = 3 + t * 3
            for j in range(3):
                c = c0 + j
                b = j
                wait_gather(c, b)
                wait_idx(c + 1, (j + 1) % 3)
                start_gather(c + 1, (j + 1) % 3)
                scale(c, b)
                wait_scatter(c - 1, (j + 2) % 3)
                start_scatter(c, b)
                load_idx(c + 2, (j + 2) % 3)
            return carry

        lax.fori_loop(0, (CHUNKS_PER_W - 5) // 3, step, 0)

        c = CHUNKS_PER_W - 2
        wait_gather(c, 0)
        wait_idx(c + 1, 1)
        start_gather(c + 1, 1)
        scale(c, 0)
        wait_scatter(c - 1, 2)
        start_scatter(c, 0)
        c = CHUNKS_PER_W - 1
        wait_gather(c, 1)
        scale(c, 1)
        wait_scatter(c - 1, 0)
        start_scatter(c, 1)
        wait_scatter(c, 1)

        plsc.subcore_barrier()

        TAIL = N - (NS - 1) * STRIPE

        @pl.when(jnp.logical_and(cid == 0, sid < NS - 1))
        def _():
            pltpu.sync_copy(agg_sh.at[pl.ds(base, STRIPE)],
                            out0_hbm.at[pl.ds(base, STRIPE)])

        @pl.when(jnp.logical_and(cid == 0, sid == NS - 1))
        def _():
            pltpu.sync_copy(agg_sh.at[pl.ds(base, TAIL)],
                            out0_hbm.at[pl.ds(base, TAIL)])

        @pl.when(jnp.logical_and(cid == 1, sid < NS - 1))
        def _():
            pltpu.sync_copy(agg_sh.at[pl.ds(base, STRIPE)],
                            out1_hbm.at[pl.ds(base, STRIPE)])

        @pl.when(jnp.logical_and(cid == 1, sid == NS - 1))
        def _():
            pltpu.sync_copy(agg_sh.at[pl.ds(base, TAIL)],
                            out1_hbm.at[pl.ds(base, TAIL)])

    return spmm_kernel(h, src2, dst2, ev2)


def _hop_body(a0_ref, a1_ref, w_ref, out_ref):
    a = a0_ref[...] + a1_ref[...]
    out_ref[...] = jnp.maximum(
        jnp.dot(a, w_ref[...], preferred_element_type=jnp.float32), 0.0
    )


def _tc_hop(a0, a1, W):
    return pl.pallas_call(
        _hop_body,
        grid=(N // ROW_BLK,),
        in_specs=[
            pl.BlockSpec((ROW_BLK, NHID), lambda i: (i, 0)),
            pl.BlockSpec((ROW_BLK, NHID), lambda i: (i, 0)),
            pl.BlockSpec((NHID, NHID), lambda i: (0, 0)),
        ],
        out_specs=pl.BlockSpec((ROW_BLK, NHID), lambda i: (i, 0)),
        out_shape=jax.ShapeDtypeStruct((N, NHID), jnp.float32),
    )(a0, a1, W)


def _final_body(h1_ref, h2_ref, h3_ref, x_ref, ax_ref, af_ref, ab_ref,
                fcw_ref, fcb_ref, out_ref):
    x = x_ref[...]
    c = jnp.sum(x * af_ref[...], axis=1, keepdims=True) + ab_ref[0, 0]
    ax = ax_ref[...]
    h1 = h1_ref[...]
    h2 = h2_ref[...]
    h3 = h3_ref[...]
    s1 = jax.nn.sigmoid(jnp.sum(h1 * ax, axis=1, keepdims=True) + c)
    s2 = jax.nn.sigmoid(jnp.sum(h2 * ax, axis=1, keepdims=True) + c)
    s3 = jax.nn.sigmoid(jnp.sum(h3 * ax, axis=1, keepdims=True) + c)
    m = jnp.maximum(jnp.maximum(s1, s2), s3)
    e1 = jnp.exp(s1 - m)
    e2 = jnp.exp(s2 - m)
    e3 = jnp.exp(s3 - m)
    denom = e1 + e2 + e3
    comb = h1 * ((e1 + e2) / denom) + h2 * (e3 / denom)
    o = jnp.dot(comb, fcw_ref[...], preferred_element_type=jnp.float32) + fcb_ref[...]
    mx = jnp.max(o, axis=1, keepdims=True)
    lse = jnp.log(jnp.sum(jnp.exp(o - mx), axis=1, keepdims=True)) + mx
    out_ref[...] = o - lse


def _tc_final(h1, h2, h3, x, ax, af, ab, fc_w, fc_b):
    row = lambda i: (i, 0)
    full = lambda i: (0, 0)
    return pl.pallas_call(
        _final_body,
        grid=(N // ROW_BLK,),
        in_specs=[
            pl.BlockSpec((ROW_BLK, NHID), row),
            pl.BlockSpec((ROW_BLK, NHID), row),
            pl.BlockSpec((ROW_BLK, NHID), row),
            pl.BlockSpec((ROW_BLK, NFEAT), row),
            pl.BlockSpec((1, NHID), full),
            pl.BlockSpec((1, NFEAT), full),
            pl.BlockSpec((1, 1), full),
            pl.BlockSpec((NHID, NCLASS), full),
            pl.BlockSpec((1, NCLASS), full),
        ],
        out_specs=pl.BlockSpec((ROW_BLK, NCLASS), row),
        out_shape=jax.ShapeDtypeStruct((N, NCLASS), jnp.float32),
    )(h1, h2, h3, x, ax, af, ab, fc_w, fc_b)


def kernel(features, edge_index, edge_vals, W0, W1, W2, att_w, att_b, fc_w, fc_b):
    dst2 = edge_index[0].reshape(NW * CHUNKS_PER_W, 1, K)
    src2 = edge_index[1].reshape(NW * CHUNKS_PER_W, 1, K)
    ev2 = edge_vals.reshape(NW * CHUNKS_PER_W, 1, K)
    Ws = [W0, W1, W2]
    hs = []
    h = features
    for i in range(HOPS):
        a0, a1 = _spmm_sc(h, src2, dst2, ev2)
        h = _tc_hop(a0, a1, Ws[i])
        hs.append(h)
    ax = att_w[:NHID, 0].reshape(1, NHID)
    af = att_w[NHID:, 0].reshape(1, NFEAT)
    ab = att_b.reshape(1, 1)
    return _tc_final(hs[0], hs[1], hs[2], features, ax, af, ab,
                     fc_w, fc_b.reshape(1, NCLASS))


# speedup vs baseline: 11.7440x; 1.2295x over previous
import functools

import jax
import jax.numpy as jnp
from jax import lax
from jax.experimental import pallas as pl
from jax.experimental.pallas import tpu as pltpu
from jax.experimental.pallas import tpu_sc as plsc

N = 10000
E = 320000
NFEAT = 128
NHID = 128
NCLASS = 64
HOPS = 3

ROW_BLK = 1000

NC = 2
NS = 16
NW = NC * NS

K = 80
EDGES_PER_W = E // NW
CHUNKS_PER_W = EDGES_PER_W // K
ZBLK = 32
NBUF = 4
IBUF = 8

STRIPE = 640

_GATHER_DNUMS = lax.GatherDimensionNumbers(
    offset_dims=(), collapsed_slice_dims=(0,), start_index_map=(0,))


def _bcast_lane(vec, ii):
    idx = jnp.full((16, 1), ii, jnp.int32)
    return lax.gather(vec, idx, _GATHER_DNUMS, (1,),
                      mode=lax.GatherScatterMode.PROMISE_IN_BOUNDS)


def _spmm_sc(h, src2, dst2, ev2):
    mesh = plsc.VectorSubcoreMesh(
        core_axis_name="c", subcore_axis_name="s", num_cores=NC, num_subcores=NS
    )

    @functools.partial(
        pl.kernel,
        out_type=(jax.ShapeDtypeStruct((N, NHID), jnp.float32),
                  jax.ShapeDtypeStruct((N, NHID), jnp.float32)),
        mesh=mesh,
        scratch_types=[
            pltpu.VMEM((IBUF, 1, K), jnp.int32),
            pltpu.VMEM((IBUF, 1, K), jnp.int32),
            pltpu.VMEM((IBUF, 1, K), jnp.float32),
            pltpu.VMEM((NBUF, K, NHID), jnp.float32),
            pltpu.VMEM((ZBLK, NHID), jnp.float32),
            pltpu.VMEM_SHARED((N, NHID), jnp.float32),
            [pltpu.SemaphoreType.DMA] * IBUF,
            [pltpu.SemaphoreType.DMA] * NBUF,
            [pltpu.SemaphoreType.DMA] * NBUF,
        ],
    )
    def spmm_kernel(h_hbm, src_hbm, dst_hbm, ev_hbm, out0_hbm, out1_hbm,
                    src_v, dst_v, ev_v, rows_v, zero_v, agg_sh,
                    isem, gsem, ssem):
        cid = lax.axis_index("c")
        sid = lax.axis_index("s")
        wid = sid * NC + cid
        cbase = wid * CHUNKS_PER_W

        zeros16 = jnp.zeros((16,), jnp.float32)

        def zrow(i, carry):
            for j in range(NHID // 16):
                zero_v[i, pl.ds(j * 16, 16)] = zeros16
            return carry

        lax.fori_loop(0, ZBLK, zrow, 0)
        base = sid * STRIPE

        @pl.when(sid < NS - 1)
        def _():
            for j in range(STRIPE // ZBLK):
                pltpu.sync_copy(zero_v, agg_sh.at[pl.ds(base + j * ZBLK, ZBLK)])

        @pl.when(sid == NS - 1)
        def _():
            for j in range(12):
                pltpu.sync_copy(zero_v, agg_sh.at[pl.ds(base + j * ZBLK, ZBLK)])
            pltpu.sync_copy(zero_v.at[pl.ds(0, 16)],
                            agg_sh.at[pl.ds(base + 12 * ZBLK, 16)])

        plsc.subcore_barrier()

        def load_idx(c, q):
            row = cbase + c
            pltpu.async_copy(src_hbm.at[row], src_v.at[q], isem[q])
            pltpu.async_copy(dst_hbm.at[row], dst_v.at[q], isem[q])
            pltpu.async_copy(ev_hbm.at[row], ev_v.at[q], isem[q])

        def wait_idx(c, q):
            row = cbase + c
            pltpu.make_async_copy(src_hbm.at[row], src_v.at[q], isem[q]).wait()
            pltpu.make_async_copy(dst_hbm.at[row], dst_v.at[q], isem[q]).wait()
            pltpu.make_async_copy(ev_hbm.at[row], ev_v.at[q], isem[q]).wait()

        def start_gather(q, b):
            pltpu.async_copy(h_hbm.at[src_v.at[q, 0]], rows_v.at[b], gsem[b])

        def wait_gather(q, b):
            pltpu.make_async_copy(h_hbm.at[src_v.at[q, 0]], rows_v.at[b],
                                  gsem[b]).wait()

        def scale(q, b):
            def grp(g, rcarry):
                evs = ev_v[q, 0, pl.ds(g * 16, 16)]
                for ii in range(16):
                    e = _bcast_lane(evs, ii)
                    r = g * 16 + ii
                    for j in range(NHID // 16):
                        sl = pl.ds(j * 16, 16)
                        rows_v[b, r, sl] = rows_v[b, r, sl] * e
                return rcarry

            lax.fori_loop(0, K // 16, grp, 0)

        def start_scatter(q, b):
            pltpu.async_copy(rows_v.at[b], agg_sh.at[dst_v.at[q, 0]],
                             ssem[b], add=True)

        def wait_scatter(q, b):
            pltpu.make_async_copy(rows_v.at[b], agg_sh.at[dst_v.at[q, 0]],
                                  ssem[b]).wait()

        for c in range(7):
            load_idx(c, c)
        wait_idx(0, 0)
        start_gather(0, 0)
        wait_idx(1, 1)
        start_gather(1, 1)
        for c in (0, 1):
            wait_gather(c, c)
            wait_idx(c + 2, c + 2)
            start_gather(c + 2, c + 2)
            scale(c, c)
            start_scatter(c, c)

        def step(t, carry):
            c0 = 2 + t * 8
            for j in range(8):
                c = c0 + j
                q = (2 + j) % IBUF
                b = (2 + j) % NBUF
                q2 = (4 + j) % IBUF
                b2 = j % NBUF
                qm2 = j % IBUF
                wait_gather(q, b)
                wait_idx(c + 2, q2)
                wait_scatter(qm2, b2)
                start_gather(q2, b2)
                scale(q, b)
                start_scatter(q, b)
                load_idx(c + 5, (7 + j) % IBUF)
            return carry

        lax.fori_loop(0, 14, step, 0)

        for c in range(114, CHUNKS_PER_W):
            q = c % IBUF
            b = c % NBUF
            wait_gather(q, b)
            if c >= 2:
                wait_scatter((c - 2) % IBUF, (c - 2) % NBUF)
            if c + 2 < CHUNKS_PER_W:
                wait_idx(c + 2, (c + 2) % IBUF)
                start_gather((c + 2) % IBUF, (c + 2) % NBUF)
            scale(q, b)
            start_scatter(q, b)
            if c + 5 < CHUNKS_PER_W:
                load_idx(c + 5, (c + 5) % IBUF)
        wait_scatter((CHUNKS_PER_W - 2) % IBUF, (CHUNKS_PER_W - 2) % NBUF)
        wait_scatter((CHUNKS_PER_W - 1) % IBUF, (CHUNKS_PER_W - 1) % NBUF)

        plsc.subcore_barrier()

        TAIL = N - (NS - 1) * STRIPE

        @pl.when(jnp.logical_and(cid == 0, sid < NS - 1))
        def _():
            pltpu.sync_copy(agg_sh.at[pl.ds(base, STRIPE)],
                            out0_hbm.at[pl.ds(base, STRIPE)])

        @pl.when(jnp.logical_and(cid == 0, sid == NS - 1))
        def _():
            pltpu.sync_copy(agg_sh.at[pl.ds(base, TAIL)],
                            out0_hbm.at[pl.ds(base, TAIL)])

        @pl.when(jnp.logical_and(cid == 1, sid < NS - 1))
        def _():
            pltpu.sync_copy(agg_sh.at[pl.ds(base, STRIPE)],
                            out1_hbm.at[pl.ds(base, STRIPE)])

        @pl.when(jnp.logical_and(cid == 1, sid == NS - 1))
        def _():
            pltpu.sync_copy(agg_sh.at[pl.ds(base, TAIL)],
                            out1_hbm.at[pl.ds(base, TAIL)])

    return spmm_kernel(h, src2, dst2, ev2)


def _hop_body(a0_ref, a1_ref, w_ref, out_ref):
    a = a0_ref[...] + a1_ref[...]
    out_ref[...] = jnp.maximum(
        jnp.dot(a, w_ref[...], preferred_element_type=jnp.float32), 0.0
    )


def _tc_hop(a0, a1, W):
    return pl.pallas_call(
        _hop_body,
        grid=(N // ROW_BLK,),
        in_specs=[
            pl.BlockSpec((ROW_BLK, NHID), lambda i: (i, 0)),
            pl.BlockSpec((ROW_BLK, NHID), lambda i: (i, 0)),
            pl.BlockSpec((NHID, NHID), lambda i: (0, 0)),
        ],
        out_specs=pl.BlockSpec((ROW_BLK, NHID), lambda i: (i, 0)),
        out_shape=jax.ShapeDtypeStruct((N, NHID), jnp.float32),
    )(a0, a1, W)


def _final_body(h1_ref, h2_ref, h3_ref, x_ref, ax_ref, af_ref, ab_ref,
                fcw_ref, fcb_ref, out_ref):
    x = x_ref[...]
    c = jnp.sum(x * af_ref[...], axis=1, keepdims=True) + ab_ref[0, 0]
    ax = ax_ref[...]
    h1 = h1_ref[...]
    h2 = h2_ref[...]
    h3 = h3_ref[...]
    s1 = jax.nn.sigmoid(jnp.sum(h1 * ax, axis=1, keepdims=True) + c)
    s2 = jax.nn.sigmoid(jnp.sum(h2 * ax, axis=1, keepdims=True) + c)
    s3 = jax.nn.sigmoid(jnp.sum(h3 * ax, axis=1, keepdims=True) + c)
    m = jnp.maximum(jnp.maximum(s1, s2), s3)
    e1 = jnp.exp(s1 - m)
    e2 = jnp.exp(s2 - m)
    e3 = jnp.exp(s3 - m)
    denom = e1 + e2 + e3
    comb = h1 * ((e1 + e2) / denom) + h2 * (e3 / denom)
    o = jnp.dot(comb, fcw_ref[...], preferred_element_type=jnp.float32) + fcb_ref[...]
    mx = jnp.max(o, axis=1, keepdims=True)
    lse = jnp.log(jnp.sum(jnp.exp(o - mx), axis=1, keepdims=True)) + mx
    out_ref[...] = o - lse


def _tc_final(h1, h2, h3, x, ax, af, ab, fc_w, fc_b):
    row = lambda i: (i, 0)
    full = lambda i: (0, 0)
    return pl.pallas_call(
        _final_body,
        grid=(N // ROW_BLK,),
        in_specs=[
            pl.BlockSpec((ROW_BLK, NHID), row),
            pl.BlockSpec((ROW_BLK, NHID), row),
            pl.BlockSpec((ROW_BLK, NHID), row),
            pl.BlockSpec((ROW_BLK, NFEAT), row),
            pl.BlockSpec((1, NHID), full),
            pl.BlockSpec((1, NFEAT), full),
            pl.BlockSpec((1, 1), full),
            pl.BlockSpec((NHID, NCLASS), full),
            pl.BlockSpec((1, NCLASS), full),
        ],
        out_specs=pl.BlockSpec((ROW_BLK, NCLASS), row),
        out_shape=jax.ShapeDtypeStruct((N, NCLASS), jnp.float32),
    )(h1, h2, h3, x, ax, af, ab, fc_w, fc_b)


def kernel(features, edge_index, edge_vals, W0, W1, W2, att_w, att_b, fc_w, fc_b):
    dst2 = edge_index[0].reshape(NW * CHUNKS_PER_W, 1, K)
    src2 = edge_index[1].reshape(NW * CHUNKS_PER_W, 1, K)
    ev2 = edge_vals.reshape(NW * CHUNKS_PER_W, 1, K)
    Ws = [W0, W1, W2]
    hs = []
    h = features
    for i in range(HOPS):
        a0, a1 = _spmm_sc(h, src2, dst2, ev2)
        h = _tc_hop(a0, a1, Ws[i])
        hs.append(h)
    ax = att_w[:NHID, 0].reshape(1, NHID)
    af = att_w[NHID:, 0].reshape(1, NFEAT)
    ab = att_b.reshape(1, 1)
    return _tc_final(hs[0], hs[1], hs[2], features, ax, af, ab,
                     fc_w, fc_b.reshape(1, NCLASS))
